# baseline (speedup 1.0000x reference)
import math

import jax
import jax.numpy as jnp
from jax import lax
from jax.experimental import pallas as pl
from jax.experimental.pallas import tpu as pltpu
from jax.experimental.pallas import tpu_sc as plsc

_NC = 2
_NS = 16
_NW = _NC * _NS

_BATCH = 16384
_SEQ = 50
_EMB = 32
_TOTAL = _BATCH * _SEQ
_SCALE = math.sqrt(float(_EMB))

_LANE = 128
_TROW = 8
_CH = _TROW * _LANE
_ROWS_PER_W = _TOTAL // _NW // _LANE
_NCHUNK = _ROWS_PER_W // _TROW
_OUT_ROWS_CH = _CH * _EMB // _LANE


def _emb_body(tok_hbm, tab_hbm, out_hbm, idx_v, rows_v, out_v, sem):
    wid = lax.axis_index("s") * _NC + lax.axis_index("c")
    tok_base = wid * _ROWS_PER_W
    out_base = wid * (_ROWS_PER_W * _EMB)

    def chunk(c, carry):
        trow = tok_base + c * _TROW
        pltpu.sync_copy(tok_hbm.at[pl.ds(trow, _TROW)], idx_v)
        cps = [
            pltpu.async_copy(
                tab_hbm.at[idx_v.at[j]],
                rows_v.at[pl.ds(j * _LANE, _LANE)],
                sem,
            )
            for j in range(_TROW)
        ]
        for cp in cps:
            cp.wait()

        def scale(i, carry2):
            for u in range(4):
                for h in range(2):
                    out_v[i, pl.ds(u * 32 + h * 16, 16)] = (
                        rows_v[i * 4 + u, pl.ds(h * 16, 16)] * _SCALE
                    )
            return carry2

        lax.fori_loop(0, _OUT_ROWS_CH, scale, 0)
        pltpu.sync_copy(
            out_v, out_hbm.at[pl.ds(out_base + c * _OUT_ROWS_CH, _OUT_ROWS_CH)]
        )
        return carry

    lax.fori_loop(0, _NCHUNK, chunk, 0)


_mesh = plsc.VectorSubcoreMesh(
    core_axis_name="c", subcore_axis_name="s", num_cores=_NC, num_subcores=_NS
)

_B_PER_W = _BATCH // _NW
_PAD_GRP = 16
_PAD_ROWS = _PAD_GRP * _SEQ * _EMB // _LANE


def _pad_body(dense_hbm, out_hbm, vm2, vm3, sem):
    wid = lax.axis_index("s") * _NC + lax.axis_index("c")
    bw = wid * _B_PER_W

    def grp(g, carry):
        b0 = bw + g * _PAD_GRP
        r0 = pl.multiple_of(b0 * (_SEQ * _EMB) // _LANE, 8)
        pltpu.sync_copy(dense_hbm.at[pl.ds(r0, _PAD_ROWS)], vm2)
        for half in range(2):
            for u in range(_PAD_GRP // 2):
                def srow(s, carry2, u=u, half=half):
                    q = (half * (_PAD_GRP // 2) + u) * _SEQ + s
                    for h in range(2):
                        vm3[u, s, pl.ds(h * 16, 16)] = vm2[
                            q >> 2, pl.ds(((q & 3) << 5) + h * 16, 16)
                        ]
                    return carry2
                lax.fori_loop(0, _SEQ, srow, 0)
            pltpu.sync_copy(
                vm3,
                out_hbm.at[pl.ds(b0 + half * (_PAD_GRP // 2), _PAD_GRP // 2), :, :],
            )
        return carry

    lax.fori_loop(0, _B_PER_W // _PAD_GRP, grp, 0)


_pad_call = pl.kernel(
    _pad_body,
    out_type=jax.ShapeDtypeStruct((_BATCH, _SEQ, _EMB), jnp.float32),
    mesh=_mesh,
    scratch_types=[
        pltpu.VMEM((_PAD_ROWS, _LANE), jnp.float32),
        pltpu.VMEM((_PAD_GRP // 2, _SEQ, _EMB), jnp.float32),
        pltpu.SemaphoreType.DMA,
    ],
    compiler_params=pltpu.CompilerParams(
        use_tc_tiling_on_sc=True, needs_layout_passes=True
    ),
)

_emb_call = pl.kernel(
    _emb_body,
    out_type=jax.ShapeDtypeStruct((_TOTAL * _EMB // _LANE, _LANE), jnp.float32),
    mesh=_mesh,
    scratch_types=[
        pltpu.VMEM((_TROW, _LANE), jnp.int32),
        pltpu.VMEM((_CH, _EMB), jnp.float32),
        pltpu.VMEM((_OUT_ROWS_CH, _LANE), jnp.float32),
        pltpu.SemaphoreType.DMA,
    ],
    compiler_params=pltpu.CompilerParams(
        use_tc_tiling_on_sc=False, needs_layout_passes=False
    ),
)


@jax.jit
def kernel(tokens, embedding):
    tok = tokens.reshape(_TOTAL // _LANE, _LANE)
    dense = _emb_call(tok, embedding)
    return _pad_call(dense)
